# Initial kernel scaffold; baseline (speedup 1.0000x reference)
import jax
import jax.numpy as jnp
from jax.experimental import pallas as pl


def kernel(tok, tau_row, k_top, col_chunk, coeff, Wq, Wk, Wv, Wq1, Wk1, Wout):
    raise NotImplementedError("write your pallas kernel here")


# speedup vs baseline: 5.5265x; 5.5265x over previous
import functools

import jax
import jax.numpy as jnp
import numpy as np
from jax import lax
from jax.experimental import pallas as pl
from jax.experimental.pallas import tpu as pltpu
from jax.experimental.pallas import tpu_sc as plsc

NBINS = 8
HEADS = 3
RANGE = 3.0
D = 768
L = 2048
DH = D // HEADS
K_TOP = 32
RB = 256
DELTA = 2.0 * RANGE / (NBINS - 1)
CENTERS = np.linspace(-RANGE, RANGE, NBINS).astype(np.float32)
NEG = -3.0e38


def _proj_body(tok_ref, wv_ref, wq_ref, wk_ref, wq1_ref, wk1_ref,
               vout_ref, uq_ref, uk_ref):
    tok_blk = tok_ref[...]
    vproj = lax.dot_general(tok_blk, wv_ref[...],
                            dimension_numbers=(((1,), (1,)), ((), ())),
                            preferred_element_type=jnp.float32)
    wq1b = wq1_ref[0, :].astype(jnp.bfloat16).astype(jnp.float32)
    wk1b = wk1_ref[0, :].astype(jnp.bfloat16).astype(jnp.float32)
    for h in range(HEADS):
        vout_ref[h] = vproj[:, h * DH:(h + 1) * DH]
        q_h = lax.dot_general(tok_blk, wq_ref[pl.ds(h * DH, DH), :],
                              dimension_numbers=(((1,), (1,)), ((), ())),
                              preferred_element_type=jnp.float32)
        k_h = lax.dot_general(tok_blk, wk_ref[pl.ds(h * DH, DH), :],
                              dimension_numbers=(((1,), (1,)), ((), ())),
                              preferred_element_type=jnp.float32)
        q_hb = q_h.astype(jnp.bfloat16).astype(jnp.float32)
        k_hb = k_h.astype(jnp.bfloat16).astype(jnp.float32)
        uq_ref[h, 0, :] = jnp.sum(q_hb * wq1b[None, :], axis=1)
        uk_ref[h, 0, :] = jnp.sum(k_hb * wk1b[None, :], axis=1)


def _proj_call(tok2, Wv, Wq, Wk, Wq1, Wk1):
    return pl.pallas_call(
        _proj_body,
        grid=(L // RB,),
        in_specs=[
            pl.BlockSpec((RB, D), lambda j: (j, 0)),
            pl.BlockSpec((D, D), lambda j: (0, 0)),
            pl.BlockSpec((D, D), lambda j: (0, 0)),
            pl.BlockSpec((D, D), lambda j: (0, 0)),
            pl.BlockSpec((1, DH), lambda j: (0, 0)),
            pl.BlockSpec((1, DH), lambda j: (0, 0)),
        ],
        out_specs=[
            pl.BlockSpec((HEADS, RB, DH), lambda j: (0, j, 0)),
            pl.BlockSpec((HEADS, 1, RB), lambda j: (0, 0, j)),
            pl.BlockSpec((HEADS, 1, RB), lambda j: (0, 0, j)),
        ],
        out_shape=[
            jax.ShapeDtypeStruct((HEADS, L, DH), jnp.float32),
            jax.ShapeDtypeStruct((HEADS, 1, L), jnp.float32),
            jax.ShapeDtypeStruct((HEADS, 1, L), jnp.float32),
        ],
    )(tok2, Wv, Wq, Wk, Wq1, Wk1)


def _score_body(uq_ref, uk_ref, tau_ref, coeff_ref, attn_ref, idx_ref):
    h = pl.program_id(0)
    uqb = uq_ref[0, 0, :]
    ukb = uk_ref[0, 0, :]
    taub = tau_ref[0, 0, :]
    diff = uqb[:, None] - ukb[None, :]
    acc = jnp.zeros((RB, L), jnp.float32)
    delta_eps = float(DELTA + 1e-06)
    for b in range(NBINS):
        cb = coeff_ref[h, b]
        dist = jnp.abs(diff - CENTERS[b])
        acc += cb * jnp.maximum(1.0 - dist / delta_eps, 0.0)
    scores = acc / (taub[:, None] + 1e-06)

    iota = lax.broadcasted_iota(jnp.int32, (RB, L), 1)
    s = scores
    vals, idxs = [], []
    for _ in range(K_TOP):
        m = jnp.max(s, axis=1)
        elig = s == m[:, None]
        ix = jnp.min(jnp.where(elig, iota, L), axis=1)
        vals.append(m)
        idxs.append(ix)
        s = jnp.where(iota == ix[:, None], NEG, s)
    tv = jnp.stack(vals, axis=1)
    ti = jnp.stack(idxs, axis=1)
    e = jnp.exp(tv - tv[:, 0:1])
    attn_ref[0] = e / jnp.sum(e, axis=1, keepdims=True)
    idx_ref[0] = ti + h * L


def _score_call(uq3, uk3, tau3, coeff):
    return pl.pallas_call(
        _score_body,
        grid=(HEADS, L // RB),
        in_specs=[
            pl.BlockSpec((1, 1, RB), lambda h, j: (h, 0, j)),
            pl.BlockSpec((1, 1, L), lambda h, j: (h, 0, 0)),
            pl.BlockSpec((1, 1, RB), lambda h, j: (0, 0, j)),
            pl.BlockSpec(memory_space=pltpu.SMEM),
        ],
        out_specs=[
            pl.BlockSpec((1, RB, K_TOP), lambda h, j: (h, j, 0)),
            pl.BlockSpec((1, RB, K_TOP), lambda h, j: (h, j, 0)),
        ],
        out_shape=[
            jax.ShapeDtypeStruct((HEADS, L, K_TOP), jnp.float32),
            jax.ShapeDtypeStruct((HEADS, L, K_TOP), jnp.int32),
        ],
    )(uq3, uk3, tau3, coeff)


def _sc_gather(vflat, gidx, attn_flat):
    info = plsc.get_sparse_core_info()
    nc, ns = info.num_cores, info.num_subcores
    nw = nc * ns
    nq = HEADS * L
    qpw = nq // nw
    mesh = plsc.VectorSubcoreMesh(core_axis_name="c", subcore_axis_name="s")

    @functools.partial(
        pl.kernel,
        out_type=jax.ShapeDtypeStruct((nq, DH), jnp.float32),
        mesh=mesh,
        scratch_types=[
            pltpu.VMEM((qpw, K_TOP), jnp.int32),
            pltpu.VMEM((qpw, K_TOP), jnp.float32),
            pltpu.VMEM((K_TOP, DH), jnp.float32),
            pltpu.VMEM((DH,), jnp.float32),
            pltpu.SemaphoreType.DMA,
        ],
    )
    def body(vflat_hbm, gidx_hbm, attn_hbm, out_hbm,
             idx_v, attn_v, rows_v, orow_v, sem):
        wid = lax.axis_index("s") * nc + lax.axis_index("c")
        qbase = wid * qpw
        pltpu.sync_copy(gidx_hbm.at[pl.ds(qbase, qpw), :], idx_v)
        pltpu.sync_copy(attn_hbm.at[pl.ds(qbase, qpw), :], attn_v)

        def one_query(q, _):
            pltpu.async_copy(vflat_hbm.at[idx_v.at[q]], rows_v, sem).wait()
            accs = [jnp.zeros((16,), jnp.float32) for _ in range(DH // 16)]
            a_vecs = [attn_v[q, pl.ds(g * 16, 16)] for g in range(K_TOP // 16)]
            for k in range(K_TOP):
                wk = a_vecs[k // 16][k % 16]
                for dd in range(DH // 16):
                    accs[dd] = accs[dd] + wk * rows_v[k, pl.ds(dd * 16, 16)]
            for dd in range(DH // 16):
                orow_v[pl.ds(dd * 16, 16)] = accs[dd]
            pltpu.sync_copy(orow_v, out_hbm.at[qbase + q])
            return ()

        lax.fori_loop(0, qpw, one_query, (), unroll=False)

    return body(vflat, gidx, attn_flat)


def _out_body(ctx_ref, wout_ref, o_ref):
    acc = jnp.zeros((RB, D), jnp.float32)
    for h in range(HEADS):
        acc += lax.dot_general(ctx_ref[h], wout_ref[:, h * DH:(h + 1) * DH],
                               dimension_numbers=(((1,), (1,)), ((), ())),
                               preferred_element_type=jnp.float32)
    o_ref[...] = acc


def _out_call(ctx3, Wout):
    return pl.pallas_call(
        _out_body,
        grid=(L // RB,),
        in_specs=[
            pl.BlockSpec((HEADS, RB, DH), lambda j: (0, j, 0)),
            pl.BlockSpec((D, D), lambda j: (0, 0)),
        ],
        out_specs=pl.BlockSpec((RB, D), lambda j: (j, 0)),
        out_shape=jax.ShapeDtypeStruct((L, D), jnp.float32),
    )(ctx3, Wout)


def kernel(tok, tau_row, k_top, col_chunk, coeff, Wq, Wk, Wv, Wq1, Wk1, Wout):
    del k_top, col_chunk
    tok2 = tok.reshape(L, D)
    tau3 = tau_row.reshape(1, 1, L)
    vflat3, uq3, uk3 = _proj_call(tok2, Wv, Wq, Wk, Wq1, Wk1)
    attn3, gidx3 = _score_call(uq3, uk3, tau3, coeff)
    ctx = _sc_gather(vflat3.reshape(HEADS * L, DH),
                     gidx3.reshape(HEADS * L, K_TOP),
                     attn3.reshape(HEADS * L, K_TOP))
    out = _out_call(ctx.reshape(HEADS, L, DH), Wout)
    return out.reshape(1, L, D)


# speedup vs baseline: 9.9802x; 1.8059x over previous
import functools

import jax
import jax.numpy as jnp
import numpy as np
from jax import lax
from jax.experimental import pallas as pl
from jax.experimental.pallas import tpu as pltpu
from jax.experimental.pallas import tpu_sc as plsc

NBINS = 8
HEADS = 3
RANGE = 3.0
D = 768
L = 2048
DH = D // HEADS
K_TOP = 32
RB = 256
DELTA = 2.0 * RANGE / (NBINS - 1)
CENTERS = np.linspace(-RANGE, RANGE, NBINS).astype(np.float32)
NEG = -3.0e38


def _proj_body(tok_ref, wv_ref, wq_ref, wk_ref, wq1_ref, wk1_ref,
               vout_ref, uq_ref, uk_ref):
    tok_blk = tok_ref[...]
    vproj = lax.dot_general(tok_blk, wv_ref[...],
                            dimension_numbers=(((1,), (1,)), ((), ())),
                            preferred_element_type=jnp.float32)
    wq1b = wq1_ref[0, :].astype(jnp.bfloat16).astype(jnp.float32)
    wk1b = wk1_ref[0, :].astype(jnp.bfloat16).astype(jnp.float32)
    for h in range(HEADS):
        vout_ref[h] = vproj[:, h * DH:(h + 1) * DH]
        q_h = lax.dot_general(tok_blk, wq_ref[pl.ds(h * DH, DH), :],
                              dimension_numbers=(((1,), (1,)), ((), ())),
                              preferred_element_type=jnp.float32)
        k_h = lax.dot_general(tok_blk, wk_ref[pl.ds(h * DH, DH), :],
                              dimension_numbers=(((1,), (1,)), ((), ())),
                              preferred_element_type=jnp.float32)
        q_hb = q_h.astype(jnp.bfloat16).astype(jnp.float32)
        k_hb = k_h.astype(jnp.bfloat16).astype(jnp.float32)
        uq_ref[h, 0, :] = jnp.sum(q_hb * wq1b[None, :], axis=1)
        uk_ref[h, 0, :] = jnp.sum(k_hb * wk1b[None, :], axis=1)


def _proj_call(tok2, Wv, Wq, Wk, Wq1, Wk1):
    return pl.pallas_call(
        _proj_body,
        grid=(L // RB,),
        in_specs=[
            pl.BlockSpec((RB, D), lambda j: (j, 0)),
            pl.BlockSpec((D, D), lambda j: (0, 0)),
            pl.BlockSpec((D, D), lambda j: (0, 0)),
            pl.BlockSpec((D, D), lambda j: (0, 0)),
            pl.BlockSpec((1, DH), lambda j: (0, 0)),
            pl.BlockSpec((1, DH), lambda j: (0, 0)),
        ],
        out_specs=[
            pl.BlockSpec((HEADS, RB, DH), lambda j: (0, j, 0)),
            pl.BlockSpec((HEADS, 1, RB), lambda j: (0, 0, j)),
            pl.BlockSpec((HEADS, 1, RB), lambda j: (0, 0, j)),
        ],
        out_shape=[
            jax.ShapeDtypeStruct((HEADS, L, DH), jnp.float32),
            jax.ShapeDtypeStruct((HEADS, 1, L), jnp.float32),
            jax.ShapeDtypeStruct((HEADS, 1, L), jnp.float32),
        ],
    )(tok2, Wv, Wq, Wk, Wq1, Wk1)


def _score_body(head_off, uq_ref, uk_ref, tau_ref, coeff_ref, attn_ref, idx_ref):
    h = 0
    uqb = uq_ref[0, 0, :]
    ukb = uk_ref[0, 0, :]
    taub = tau_ref[0, 0, :]
    diff = uqb[:, None] - ukb[None, :]
    acc = jnp.zeros((RB, L), jnp.float32)
    delta_eps = float(DELTA + 1e-06)
    for b in range(NBINS):
        cb = coeff_ref[h, b]
        dist = jnp.abs(diff - CENTERS[b])
        acc += cb * jnp.maximum(1.0 - dist / delta_eps, 0.0)
    scores = acc / (taub[:, None] + 1e-06)

    iota_f = lax.broadcasted_iota(jnp.int32, (RB, L), 1).astype(jnp.float32)
    s = scores
    vals, idxs = [], []
    for _ in range(K_TOP):
        m = jnp.max(s, axis=1)
        elig = s == m[:, None]
        ix = jnp.min(jnp.where(elig, iota_f, float(L)), axis=1)
        vals.append(m)
        idxs.append(ix)
        s = jnp.where(iota_f == ix[:, None], NEG, s)
    tv = jnp.stack(vals, axis=1)
    ti = jnp.stack(idxs, axis=1).astype(jnp.int32)
    e = jnp.exp(tv - tv[:, 0:1])
    attn_ref[0] = e / jnp.sum(e, axis=1, keepdims=True)
    idx_ref[0] = ti + head_off


def _score_call(uq_h, uk_h, tau_h, coeff_h, head_off):
    nrows = uq_h.shape[2]
    return pl.pallas_call(
        functools.partial(_score_body, head_off),
        grid=(nrows // RB,),
        in_specs=[
            pl.BlockSpec((1, 1, RB), lambda j: (0, 0, j)),
            pl.BlockSpec((1, 1, L), lambda j: (0, 0, 0)),
            pl.BlockSpec((1, 1, RB), lambda j: (0, 0, j)),
            pl.BlockSpec(memory_space=pltpu.SMEM),
        ],
        out_specs=[
            pl.BlockSpec((1, RB, K_TOP), lambda j: (0, j, 0)),
            pl.BlockSpec((1, RB, K_TOP), lambda j: (0, j, 0)),
        ],
        out_shape=[
            jax.ShapeDtypeStruct((1, nrows, K_TOP), jnp.float32),
            jax.ShapeDtypeStruct((1, nrows, K_TOP), jnp.int32),
        ],
    )(uq_h, uk_h, tau_h, coeff_h)


def _sc_gather(vflat, gidx, attn_flat):
    info = plsc.get_sparse_core_info()
    nc, ns = info.num_cores, info.num_subcores
    nw = nc * ns
    nq = gidx.shape[0]
    qpw = nq // nw
    mesh = plsc.VectorSubcoreMesh(core_axis_name="c", subcore_axis_name="s")

    @functools.partial(
        pl.kernel,
        out_type=jax.ShapeDtypeStruct((nq, DH), jnp.float32),
        mesh=mesh,
        scratch_types=[
            pltpu.VMEM((qpw, K_TOP), jnp.int32),
            pltpu.VMEM((qpw, K_TOP), jnp.float32),
            pltpu.VMEM((2, K_TOP, DH), jnp.float32),
            pltpu.VMEM((qpw, DH), jnp.float32),
            pltpu.SemaphoreType.DMA,
            pltpu.SemaphoreType.DMA,
        ],
    )
    def body(vflat_hbm, gidx_hbm, attn_hbm, out_hbm,
             idx_v, attn_v, rows_v, out_v, sem0, sem1):
        wid = lax.axis_index("s") * nc + lax.axis_index("c")
        qbase = wid * qpw
        pltpu.sync_copy(gidx_hbm.at[pl.ds(qbase, qpw), :], idx_v)
        pltpu.sync_copy(attn_hbm.at[pl.ds(qbase, qpw), :], attn_v)
        sems = (sem0, sem1)
        nbuf = 2

        def gather(q, buf):
            return pltpu.async_copy(vflat_hbm.at[idx_v.at[q]],
                                    rows_v.at[buf], sems[buf])

        for b in range(nbuf):
            gather(b, b)

        def one_query(qq, _):
            for b in range(nbuf):
                q = qq + b
                pltpu.make_async_copy(vflat_hbm.at[idx_v.at[q]],
                                      rows_v.at[b], sems[b]).wait()
                accs = [jnp.zeros((16,), jnp.float32)
                        for _ in range(DH // 16)]
                a_vecs = [attn_v[q, pl.ds(g * 16, 16)]
                          for g in range(K_TOP // 16)]
                for k in range(K_TOP):
                    wk = a_vecs[k // 16][k % 16]
                    for dd in range(DH // 16):
                        accs[dd] = accs[dd] + wk * rows_v[b, k,
                                                          pl.ds(dd * 16, 16)]
                for dd in range(DH // 16):
                    out_v[q, pl.ds(dd * 16, 16)] = accs[dd]

                @pl.when(q + nbuf < qpw)
                def _():
                    gather(q + nbuf, b)
            return ()

        lax.fori_loop(0, qpw // nbuf, lambda t, c: one_query(nbuf * t, c), (),
                      unroll=False)
        pltpu.sync_copy(out_v, out_hbm.at[pl.ds(qbase, qpw), :])

    return body(vflat, gidx, attn_flat)


def _out_body(c0_ref, c1_ref, c2_ref, wout_ref, o_ref):
    acc = jnp.zeros((RB, D), jnp.float32)
    for h, c_ref in enumerate((c0_ref, c1_ref, c2_ref)):
        acc += lax.dot_general(c_ref[...], wout_ref[:, h * DH:(h + 1) * DH],
                               dimension_numbers=(((1,), (1,)), ((), ())),
                               preferred_element_type=jnp.float32)
    o_ref[...] = acc


def _out_call(ctx_parts, Wout):
    return pl.pallas_call(
        _out_body,
        grid=(L // RB,),
        in_specs=[
            pl.BlockSpec((RB, DH), lambda j: (j, 0)),
            pl.BlockSpec((RB, DH), lambda j: (j, 0)),
            pl.BlockSpec((RB, DH), lambda j: (j, 0)),
            pl.BlockSpec((D, D), lambda j: (0, 0)),
        ],
        out_specs=pl.BlockSpec((RB, D), lambda j: (j, 0)),
        out_shape=jax.ShapeDtypeStruct((L, D), jnp.float32),
    )(*ctx_parts, Wout)


def kernel(tok, tau_row, k_top, col_chunk, coeff, Wq, Wk, Wv, Wq1, Wk1, Wout):
    del k_top, col_chunk
    tok2 = tok.reshape(L, D)
    tau3 = tau_row.reshape(1, 1, L)
    vflat3, uq3, uk3 = _proj_call(tok2, Wv, Wq, Wk, Wq1, Wk1)
    vflat = vflat3.reshape(HEADS * L, DH)
    ctx_parts = []
    for h in range(HEADS):
        attn_h, gidx_h = _score_call(uq3[h:h + 1], uk3[h:h + 1], tau3,
                                     coeff[h:h + 1], h * L)
        ctx_parts.append(_sc_gather(vflat, gidx_h.reshape(L, K_TOP),
                                    attn_h.reshape(L, K_TOP)))
    out = _out_call(ctx_parts, Wout)
    return out.reshape(1, L, D)
